# baseline (speedup 1.0000x reference)
import jax
import jax.numpy as jnp
from jax import lax
from jax.experimental import pallas as pl
from jax.experimental.pallas import tpu as pltpu, tpu_sc as plsc

_NC = 2
_NS = 16
_NW = _NC * _NS
_L = 16

_BATCH = 16384
_DIM = 32
_NROWS = 1000000
_BW = _BATCH // _NW
_G = 4


def _rec_mf_body(users_hbm, items_hbm, u_tab_hbm, i_tab_hbm, out_hbm,
                 idx_us, idx_is, ublk, iblk, out_v, sem_a, sem_b):
    wid = lax.axis_index("s") * _NC + lax.axis_index("c")
    base = wid * _BW

    pltpu.sync_copy(users_hbm.at[pl.ds(base, _BW)], idx_us)
    pltpu.sync_copy(items_hbm.at[pl.ds(base, _BW)], idx_is)

    lane_iota = lax.iota(jnp.int32, _L)
    lo_rows = lane_iota
    hi_rows = lane_iota + _L

    def tile_group(t, _):
        uvec = idx_us[pl.ds(t * _L, _L)]
        ivec = idx_is[pl.ds(t * _L, _L)]
        bu_vec = uvec & -128
        bi_vec = ivec & -128
        lu_vec = uvec - bu_vec
        li_vec = ivec - bi_vec
        sems = (sem_a, sem_b)

        def fire(p):
            copies = []
            for r in range(_G):
                ln = p * _G + r
                bu = pl.multiple_of(bu_vec[ln], 128)
                bi = pl.multiple_of(bi_vec[ln], 128)
                copies.append(pltpu.async_copy(
                    u_tab_hbm.at[:, pl.ds(bu, 128)], ublk.at[p & 1, r],
                    sems[p & 1]))
                copies.append(pltpu.async_copy(
                    i_tab_hbm.at[:, pl.ds(bi, 128)], iblk.at[p & 1, r],
                    sems[p & 1]))
            return copies

        acc = jnp.zeros((_L,), jnp.float32)
        inflight = {0: fire(0), 1: fire(1)}
        for p in range(_L // _G):
            for cp in inflight.pop(p):
                cp.wait()
            if p + 2 < _L // _G:
                inflight[p + 2] = fire(p + 2)
            for r in range(_G):
                ln = p * _G + r
                lu_v = jnp.full((_L,), lu_vec[ln], jnp.int32)
                li_v = jnp.full((_L,), li_vec[ln], jnp.int32)
                a_lo = plsc.load_gather(ublk.at[p & 1, r], [lo_rows, lu_v])
                a_hi = plsc.load_gather(ublk.at[p & 1, r], [hi_rows, lu_v])
                b_lo = plsc.load_gather(iblk.at[p & 1, r], [lo_rows, li_v])
                b_hi = plsc.load_gather(iblk.at[p & 1, r], [hi_rows, li_v])
                s = a_lo * b_lo + a_hi * b_hi
                acc = acc + jnp.where(lane_iota == ln,
                                      jnp.sum(s, axis=0), 0.0)
        out_v[pl.ds(t * _L, _L)] = 1.0 / (1.0 + jnp.exp(-acc))
        return 0

    lax.fori_loop(0, _BW // _L, tile_group, 0)

    pltpu.sync_copy(out_v, out_hbm.at[pl.ds(base, _BW)])


@jax.jit
def kernel(users, items, user_table, item_table):
    mesh = plsc.VectorSubcoreMesh(
        core_axis_name="c", subcore_axis_name="s",
        num_cores=_NC, num_subcores=_NS)
    f = pl.kernel(
        _rec_mf_body,
        out_type=jax.ShapeDtypeStruct((_BATCH,), jnp.float32),
        mesh=mesh,
        compiler_params=pltpu.CompilerParams(needs_layout_passes=False),
        scratch_types=[
            pltpu.VMEM((_BW,), jnp.int32),
            pltpu.VMEM((_BW,), jnp.int32),
            pltpu.VMEM((2, _G, _DIM, 128), jnp.float32),
            pltpu.VMEM((2, _G, _DIM, 128), jnp.float32),
            pltpu.VMEM((_BW,), jnp.float32),
            pltpu.SemaphoreType.DMA,
            pltpu.SemaphoreType.DMA,
        ],
    )
    return f(users, items, user_table.T, item_table.T)


# speedup vs baseline: 1.0223x; 1.0223x over previous
import jax
import jax.numpy as jnp
from jax import lax
from jax.experimental import pallas as pl
from jax.experimental.pallas import tpu as pltpu, tpu_sc as plsc

_NC = 2
_NS = 16
_NW = _NC * _NS
_L = 16

_BATCH = 16384
_DIM = 32
_NROWS = 1000000
_BW = _BATCH // _NW
_G = 8


def _rec_mf_body(users_hbm, items_hbm, u_tab_hbm, i_tab_hbm, out_hbm,
                 idx_us, idx_is, ublk, iblk, out_v, sem):
    wid = lax.axis_index("s") * _NC + lax.axis_index("c")
    base = wid * _BW

    pltpu.sync_copy(users_hbm.at[pl.ds(base, _BW)], idx_us)
    pltpu.sync_copy(items_hbm.at[pl.ds(base, _BW)], idx_is)

    lane_iota = lax.iota(jnp.int32, _L)
    lo_rows = lane_iota
    hi_rows = lane_iota + _L

    def tile_group(t, _):
        uvec = idx_us[pl.ds(t * _L, _L)]
        ivec = idx_is[pl.ds(t * _L, _L)]
        bu_vec = uvec & -128
        bi_vec = ivec & -128
        lu_vec = uvec - bu_vec
        li_vec = ivec - bi_vec
        acc = jnp.zeros((_L,), jnp.float32)
        for p in range(_L // _G):
            lanes = []
            copies = []
            for r in range(_G):
                ln = p * _G + r
                bu = pl.multiple_of(bu_vec[ln], 128)
                bi = pl.multiple_of(bi_vec[ln], 128)
                lanes.append((lu_vec[ln], li_vec[ln]))
                copies.append(pltpu.async_copy(
                    u_tab_hbm.at[:, pl.ds(bu, 128)], ublk.at[r], sem))
                copies.append(pltpu.async_copy(
                    i_tab_hbm.at[:, pl.ds(bi, 128)], iblk.at[r], sem))
            for cp in copies:
                cp.wait()
            for r in range(_G):
                lu, li = lanes[r]
                lu_v = jnp.full((_L,), lu, jnp.int32)
                li_v = jnp.full((_L,), li, jnp.int32)
                a_lo = plsc.load_gather(ublk.at[r], [lo_rows, lu_v])
                a_hi = plsc.load_gather(ublk.at[r], [hi_rows, lu_v])
                b_lo = plsc.load_gather(iblk.at[r], [lo_rows, li_v])
                b_hi = plsc.load_gather(iblk.at[r], [hi_rows, li_v])
                s = a_lo * b_lo + a_hi * b_hi
                acc = acc + jnp.where(lane_iota == p * _G + r,
                                      jnp.sum(s, axis=0), 0.0)
        out_v[pl.ds(t * _L, _L)] = 1.0 / (1.0 + jnp.exp(-acc))
        return 0

    lax.fori_loop(0, _BW // _L, tile_group, 0)

    pltpu.sync_copy(out_v, out_hbm.at[pl.ds(base, _BW)])


@jax.jit
def kernel(users, items, user_table, item_table):
    mesh = plsc.VectorSubcoreMesh(
        core_axis_name="c", subcore_axis_name="s",
        num_cores=_NC, num_subcores=_NS)
    f = pl.kernel(
        _rec_mf_body,
        out_type=jax.ShapeDtypeStruct((_BATCH,), jnp.float32),
        mesh=mesh,
        compiler_params=pltpu.CompilerParams(needs_layout_passes=False),
        scratch_types=[
            pltpu.VMEM((_BW,), jnp.int32),
            pltpu.VMEM((_BW,), jnp.int32),
            pltpu.VMEM((_G, _DIM, 128), jnp.float32),
            pltpu.VMEM((_G, _DIM, 128), jnp.float32),
            pltpu.VMEM((_BW,), jnp.float32),
            pltpu.SemaphoreType.DMA,
        ],
    )
    return f(users, items, user_table.T, item_table.T)
